# baseline (speedup 1.0000x reference)
import jax
import jax.numpy as jnp
from jax import lax
from jax.experimental import pallas as pl

N_VOCAB = 2048
SEQ_BLK = 2048
SEQ_OUT = 8193
N_BLKS = 5


def _onehot_block(ids_ref, out_ref):
    ids = ids_ref[0, 0, 0, :]
    cls = lax.broadcasted_iota(jnp.int32, (SEQ_BLK, N_VOCAB), 1)
    out_ref[0] = (ids[:, None] == cls).astype(jnp.float32)


def kernel(input_ids):
    batch, seq = input_ids.shape
    padded = jnp.pad(
        input_ids.astype(jnp.int32),
        ((0, 0), (1, N_BLKS * SEQ_BLK - seq - 1)),
        constant_values=0,
    )
    ids4 = padded.reshape(batch, N_BLKS, 1, SEQ_BLK)

    return pl.pallas_call(
        _onehot_block,
        grid=(batch, N_BLKS),
        in_specs=[
            pl.BlockSpec((1, 1, 1, SEQ_BLK), lambda b, j: (b, j, 0, 0)),
        ],
        out_specs=pl.BlockSpec((1, SEQ_BLK, N_VOCAB), lambda b, j: (b, j, 0)),
        out_shape=jax.ShapeDtypeStruct((batch, SEQ_OUT, N_VOCAB), jnp.float32),
    )(ids4)


# speedup vs baseline: 1.0236x; 1.0236x over previous
import jax
import jax.numpy as jnp
from jax import lax
from jax.experimental import pallas as pl
from jax.experimental.pallas import tpu as pltpu

N_VOCAB = 2048
CHUNK = 512
SEQ = 8192
SEQ_OUT = 8193
N_CHUNKS = SEQ // CHUNK
K_BUF = 4


def _onehot_manual(ids_ref, out_ref, *scratch):
    bufs = scratch[:K_BUF]
    sems = scratch[K_BUF:2 * K_BUF]
    last_sem = scratch[2 * K_BUF]

    cls = lax.broadcasted_iota(jnp.int32, (CHUNK, N_VOCAB), 1)
    copies = []
    i = 0
    for b in range(4):
        for c in range(N_CHUNKS):
            k = i % K_BUF
            if i >= K_BUF:
                copies[i - K_BUF].wait()
            ids = ids_ref[b, pl.ds(c * CHUNK, CHUNK)]
            bufs[k][...] = (ids[:, None] == cls).astype(jnp.float32)
            cp = pltpu.make_async_copy(
                bufs[k], out_ref.at[b, pl.ds(c * CHUNK, CHUNK), :], sems[k]
            )
            cp.start()
            copies.append(cp)
            i += 1
    for b in range(4):
        k = i % K_BUF
        if i >= K_BUF:
            copies[i - K_BUF].wait()
        ids_tail = ids_ref[b, pl.ds(SEQ, 8)]
        bufs[k][pl.ds(0, 8), :] = (ids_tail[:, None] == cls[:8, :]).astype(
            jnp.float32
        )
        cp = pltpu.make_async_copy(
            bufs[k].at[pl.ds(0, 1), :],
            out_ref.at[b, pl.ds(SEQ, 1), :],
            last_sem,
        )
        cp.start()
        copies.append(cp)
        i += 1
    for cp in copies[i - K_BUF:]:
        cp.wait()


def kernel(input_ids):
    batch, seq = input_ids.shape
    padded = jnp.pad(
        input_ids.astype(jnp.int32), ((0, 0), (1, 511)), constant_values=0
    )

    return pl.pallas_call(
        _onehot_manual,
        in_specs=[pl.BlockSpec(memory_space=pltpu.VMEM)],
        out_specs=pl.BlockSpec(memory_space=pl.ANY),
        out_shape=jax.ShapeDtypeStruct((batch, SEQ_OUT, N_VOCAB), jnp.float32),
        scratch_shapes=(
            [pltpu.VMEM((CHUNK, N_VOCAB), jnp.float32)] * K_BUF
            + [pltpu.SemaphoreType.DMA] * K_BUF
            + [pltpu.SemaphoreType.DMA]
        ),
    )(padded)
